# baseline (speedup 1.0000x reference)
import functools

import jax
import jax.numpy as jnp
from jax import lax
from jax.experimental import pallas as pl
from jax.experimental.pallas import tpu as pltpu
from jax.experimental.pallas import tpu_sc as plsc

_D = 128
_BT = 4 * 8192
_T = 8192
_ND = 1024

_NC, _NS = 2, 16
_NW = _NC * _NS
_B_PER_W = _BT // _NW
_HALF = _B_PER_W // 2
_CG = 64
_N_CHUNK = _HALF // _CG


def _sc_gather_pack(idx_hbm, table_hbm, z_hbm, idx_v, gbuf, zbuf,
                    sg0a, sg0b, sg1a, sg1b, sz0, sz1):
    wid = lax.axis_index("s") * _NC + lax.axis_index("c")
    zbase = wid * _HALF
    pltpu.sync_copy(idx_hbm.at[wid], idx_v)
    gsems = ((sg0a, sg0b), (sg1a, sg1b))
    zsems = (sz0, sz1)

    def start_gathers(c):
        buf = c % 2
        ca = pltpu.async_copy(
            table_hbm.at[idx_v.at[pl.ds(c * _CG, _CG)]],
            gbuf.at[buf].at[pl.ds(0, _CG)], gsems[buf][0])
        cb = pltpu.async_copy(
            table_hbm.at[idx_v.at[pl.ds(_HALF + c * _CG, _CG)]],
            gbuf.at[buf].at[pl.ds(_CG, _CG)], gsems[buf][1])
        return ca, cb

    pend_g = [None, None]
    pend_z = [None, None]
    pend_g[0] = start_gathers(0)

    for c in range(_N_CHUNK):
        buf = c % 2
        if c + 1 < _N_CHUNK:
            pend_g[(c + 1) % 2] = start_gathers(c + 1)
        pend_g[buf][0].wait()
        pend_g[buf][1].wait()
        if pend_z[buf] is not None:
            pend_z[buf].wait()

        gb = gbuf.at[buf]
        zb = zbuf.at[buf]

        def body(s, carry):
            for v in range(8):
                sl = pl.ds(v * 16, 16)
                au = gb[s, sl] + jnp.uint32(0x8000)
                bu = gb[_CG + s, sl] + jnp.uint32(0x8000)
                zb[s, sl] = (au >> jnp.uint32(16)) | (bu & jnp.uint32(0xFFFF0000))
            return carry

        lax.fori_loop(0, _CG, body, 0)
        pend_z[buf] = pltpu.async_copy(
            zb, z_hbm.at[pl.ds(zbase + c * _CG, _CG)], zsems[buf])

    for p in pend_z:
        if p is not None:
            p.wait()


def _make_gather_pack():
    mesh = plsc.VectorSubcoreMesh(core_axis_name="c", subcore_axis_name="s")
    return pl.kernel(
        _sc_gather_pack,
        out_type=jax.ShapeDtypeStruct((_BT // 2, _D), jnp.uint32),
        mesh=mesh,
        scratch_types=[
            pltpu.VMEM((_B_PER_W,), jnp.int32),
            pltpu.VMEM((2, 2 * _CG, _D), jnp.uint32),
            pltpu.VMEM((2, _CG, _D), jnp.uint32),
            pltpu.SemaphoreType.DMA,
            pltpu.SemaphoreType.DMA,
            pltpu.SemaphoreType.DMA,
            pltpu.SemaphoreType.DMA,
            pltpu.SemaphoreType.DMA,
            pltpu.SemaphoreType.DMA,
        ],
    )


_TB = 1024


def _tc_matmul(z_ref, wpe_ref, w_ref, b_ref, out_ref):
    z = z_ref[...]
    a = lax.bitcast_convert_type(z << jnp.uint32(16), jnp.float32)
    bb = lax.bitcast_convert_type(z & jnp.uint32(0xFFFF0000), jnp.float32)
    ya = (a + wpe_ref[0:_HALF]).astype(jnp.bfloat16)
    yb = (bb + wpe_ref[_HALF:_TB]).astype(jnp.bfloat16)
    out_ref[0:_HALF] = (
        jnp.dot(ya, w_ref[...], preferred_element_type=jnp.float32) + b_ref[...]
    )
    out_ref[_HALF:_TB] = (
        jnp.dot(yb, w_ref[...], preferred_element_type=jnp.float32) + b_ref[...]
    )


def _make_matmul():
    nwpe = _T // _TB
    grid = (nwpe, _BT // _T)
    return pl.pallas_call(
        _tc_matmul,
        grid=grid,
        in_specs=[
            pl.BlockSpec((_HALF, _D), lambda j, k: (k * nwpe + j, 0)),
            pl.BlockSpec((_TB, _D), lambda j, k: (j, 0)),
            pl.BlockSpec((_D, _ND), lambda j, k: (0, 0)),
            pl.BlockSpec((1, _ND), lambda j, k: (0, 0)),
        ],
        out_specs=pl.BlockSpec((_TB, _ND), lambda j, k: (k * nwpe + j, 0)),
        out_shape=jax.ShapeDtypeStruct((_BT, _ND), jnp.float32),
    )


@jax.jit
def kernel(x, wte, wpe, W, b):
    Bsz, T = x.shape
    idx = x.reshape(_NW, _B_PER_W)
    wte_u32 = lax.bitcast_convert_type(wte, jnp.uint32)
    z = _make_gather_pack()(idx, wte_u32)
    out = _make_matmul()(z, wpe, W.astype(jnp.bfloat16), b.reshape(1, _ND))
    return out.reshape(Bsz, T, _ND)
